# baseline (speedup 1.0000x reference)
import functools

import jax
import jax.numpy as jnp
from jax import lax
from jax.experimental import pallas as pl
from jax.experimental.pallas import tpu as pltpu
from jax.experimental.pallas import tpu_sc as plsc

N = 100000
C = 64
LANES = 16
NUM_CORES = 2
NUM_SUBCORES = 16
NW = NUM_CORES * NUM_SUBCORES

ROWS_MAIN = 3200
ROWS_LAST = 800
CHUNK_ROWS = 128
GROUPS_PER_CHUNK = CHUNK_ROWS // LANES
CHUNKS_MAIN = ROWS_MAIN // CHUNK_ROWS
CHUNKS_LAST = ROWS_LAST // CHUNK_ROWS
TAIL_ROWS = ROWS_LAST - CHUNKS_LAST * CHUNK_ROWS
TAIL_GROUPS = TAIL_ROWS // LANES


def _consume(cand_buf, wtab, cbase, n_groups, out_row0, ow_buf, oc_buf):
    for g in range(n_groups):
        base = g * LANES
        best_c = cand_buf[cbase, pl.ds(base, LANES)]
        best_w = plsc.load_gather(wtab, [best_c])
        for c in range(1, C):
            cand = cand_buf[cbase + c, pl.ds(base, LANES)]
            w = plsc.load_gather(wtab, [cand])
            upd = w > best_w
            best_c = jnp.where(upd, cand, best_c)
            best_w = jnp.maximum(best_w, w)
        row = out_row0 + g * LANES
        ow_buf[pl.ds(row, LANES)] = best_w
        oc_buf[pl.ds(row, LANES)] = best_c


@functools.partial(
    pl.kernel,
    out_type=(
        jax.ShapeDtypeStruct((N,), jnp.float32),
        jax.ShapeDtypeStruct((N,), jnp.int32),
    ),
    mesh=plsc.VectorSubcoreMesh(
        core_axis_name="c", subcore_axis_name="s",
        num_cores=NUM_CORES, num_subcores=NUM_SUBCORES,
    ),
    scratch_types=[
        pltpu.VMEM((N,), jnp.float32),
        pltpu.VMEM((2 * C, CHUNK_ROWS), jnp.int32),
        pltpu.VMEM((TAIL_ROWS * C,), jnp.int32),
        pltpu.VMEM((ROWS_MAIN,), jnp.float32),
        pltpu.VMEM((ROWS_MAIN,), jnp.int32),
        pltpu.SemaphoreType.DMA,
    ],
    compiler_params=pltpu.CompilerParams(needs_layout_passes=False),
)
def _sc_max_select(w_hbm, cand_hbm, tail_hbm, out_w_hbm, out_c_hbm,
                   wtab, cand_buf, tail_buf, ow_buf, oc_buf, sem):
    wid = lax.axis_index("s") * NUM_CORES + lax.axis_index("c")
    wbase = wid * ROWS_MAIN
    last = wid == NW - 1
    n_chunks = jnp.where(last, CHUNKS_LAST, CHUNKS_MAIN)

    pltpu.sync_copy(w_hbm, wtab)

    pltpu.sync_copy(cand_hbm.at[:, pl.ds(wbase, CHUNK_ROWS)],
                    cand_buf.at[pl.ds(0, C)])

    def chunk_body(j, carry):
        nxt = jnp.minimum(j + 1, n_chunks - 1)
        p_next = ((j + 1) & 1) * C
        cp = pltpu.async_copy(
            cand_hbm.at[:, pl.ds(wbase + nxt * CHUNK_ROWS, CHUNK_ROWS)],
            cand_buf.at[pl.ds(p_next, C)], sem)
        _consume(cand_buf, wtab, (j & 1) * C, GROUPS_PER_CHUNK,
                 j * CHUNK_ROWS, ow_buf, oc_buf)
        cp.wait()
        return carry

    lax.fori_loop(0, n_chunks, chunk_body, 0)

    @pl.when(last)
    def _():
        pltpu.sync_copy(tail_hbm, tail_buf)
        lane = lax.iota(jnp.int32, LANES)
        for g in range(TAIL_GROUPS):
            rowoff = (lane + g * LANES) * C
            best_c = plsc.load_gather(tail_buf, [rowoff])
            best_w = plsc.load_gather(wtab, [best_c])

            def tail_col(c, carry):
                bc, bw = carry
                cand = plsc.load_gather(tail_buf, [rowoff + c])
                w = plsc.load_gather(wtab, [cand])
                upd = w > bw
                return (jnp.where(upd, cand, bc), jnp.maximum(bw, w))

            best_c, best_w = lax.fori_loop(1, C, tail_col, (best_c, best_w))
            row = CHUNKS_LAST * CHUNK_ROWS + g * LANES
            ow_buf[pl.ds(row, LANES)] = best_w
            oc_buf[pl.ds(row, LANES)] = best_c

    pltpu.sync_copy(ow_buf.at[pl.ds(0, ROWS_LAST)],
                    out_w_hbm.at[pl.ds(wbase, ROWS_LAST)])
    pltpu.sync_copy(oc_buf.at[pl.ds(0, ROWS_LAST)],
                    out_c_hbm.at[pl.ds(wbase, ROWS_LAST)])

    @pl.when(jnp.logical_not(last))
    def _():
        extra = ROWS_MAIN - ROWS_LAST
        pltpu.sync_copy(ow_buf.at[pl.ds(ROWS_LAST, extra)],
                        out_w_hbm.at[pl.ds(wbase + ROWS_LAST, extra)])
        pltpu.sync_copy(oc_buf.at[pl.ds(ROWS_LAST, extra)],
                        out_c_hbm.at[pl.ds(wbase + ROWS_LAST, extra)])


def kernel(weights, candidates):
    w = weights.reshape(N).astype(jnp.float32)
    cand = candidates.astype(jnp.int32)
    cand_t = cand.T
    tail = cand[N - TAIL_ROWS:, :].reshape(TAIL_ROWS * C)
    max_w, max_c = _sc_max_select(w, cand_t, tail)
    return (max_w.reshape(N, 1), max_c.astype(candidates.dtype))


# speedup vs baseline: 1.0126x; 1.0126x over previous
import functools

import jax
import jax.numpy as jnp
from jax import lax
from jax.experimental import pallas as pl
from jax.experimental.pallas import tpu as pltpu
from jax.experimental.pallas import tpu_sc as plsc

N = 100000
C = 64
LANES = 16
NUM_CORES = 2
NUM_SUBCORES = 16
NW = NUM_CORES * NUM_SUBCORES

ROWS_MAIN = 3200
ROWS_LAST = 800
CHUNK_ROWS = 128
GROUPS_PER_CHUNK = CHUNK_ROWS // LANES
CHUNKS_MAIN = ROWS_MAIN // CHUNK_ROWS
CHUNKS_LAST = ROWS_LAST // CHUNK_ROWS
TAIL_ROWS = ROWS_LAST - CHUNKS_LAST * CHUNK_ROWS
TAIL_GROUPS = TAIL_ROWS // LANES


def _consume(cand_buf, wtab, cbase, n_groups, out_row0, ow_buf, oc_buf):
    for g in range(n_groups):
        base = g * LANES
        best_c = cand_buf[cbase, pl.ds(base, LANES)]
        best_w = plsc.load_gather(wtab, [best_c])
        for c in range(1, C):
            cand = cand_buf[cbase + c, pl.ds(base, LANES)]
            w = plsc.load_gather(wtab, [cand])
            upd = w > best_w
            best_c = jnp.where(upd, cand, best_c)
            best_w = jnp.maximum(best_w, w)
        row = out_row0 + g * LANES
        ow_buf[pl.ds(row, LANES)] = best_w
        oc_buf[pl.ds(row, LANES)] = best_c


@functools.partial(
    pl.kernel,
    out_type=(
        jax.ShapeDtypeStruct((N,), jnp.float32),
        jax.ShapeDtypeStruct((N,), jnp.int32),
    ),
    mesh=plsc.VectorSubcoreMesh(
        core_axis_name="c", subcore_axis_name="s",
        num_cores=NUM_CORES, num_subcores=NUM_SUBCORES,
    ),
    scratch_types=[
        pltpu.VMEM((N,), jnp.float32),
        pltpu.VMEM((2 * C, CHUNK_ROWS), jnp.int32),
        pltpu.VMEM((TAIL_ROWS * C,), jnp.int32),
        pltpu.VMEM((ROWS_MAIN,), jnp.float32),
        pltpu.VMEM((ROWS_MAIN,), jnp.int32),
        pltpu.SemaphoreType.DMA,
    ],
    compiler_params=pltpu.CompilerParams(needs_layout_passes=False),
)
def _sc_max_select(w_hbm, cand_hbm, tail_hbm, out_w_hbm, out_c_hbm,
                   wtab, cand_buf, tail_buf, ow_buf, oc_buf, sem):
    wid = lax.axis_index("s") * NUM_CORES + lax.axis_index("c")
    wbase = wid * ROWS_MAIN
    last = wid == NW - 1
    n_chunks = jnp.where(last, CHUNKS_LAST, CHUNKS_MAIN)

    prime = pltpu.async_copy(cand_hbm.at[:, pl.ds(wbase, CHUNK_ROWS)],
                             cand_buf.at[pl.ds(0, C)], sem)
    pltpu.sync_copy(w_hbm, wtab)
    prime.wait()


    def chunk_body(j, carry):
        nxt = jnp.minimum(j + 1, n_chunks - 1)
        p_next = ((j + 1) & 1) * C
        cp = pltpu.async_copy(
            cand_hbm.at[:, pl.ds(wbase + nxt * CHUNK_ROWS, CHUNK_ROWS)],
            cand_buf.at[pl.ds(p_next, C)], sem)
        _consume(cand_buf, wtab, (j & 1) * C, GROUPS_PER_CHUNK,
                 j * CHUNK_ROWS, ow_buf, oc_buf)
        cp.wait()
        return carry

    lax.fori_loop(0, n_chunks, chunk_body, 0)

    @pl.when(last)
    def _():
        pltpu.sync_copy(tail_hbm, tail_buf)
        lane = lax.iota(jnp.int32, LANES)
        for g in range(TAIL_GROUPS):
            rowoff = (lane + g * LANES) * C
            best_c = plsc.load_gather(tail_buf, [rowoff])
            best_w = plsc.load_gather(wtab, [best_c])

            def tail_col(c, carry):
                bc, bw = carry
                cand = plsc.load_gather(tail_buf, [rowoff + c])
                w = plsc.load_gather(wtab, [cand])
                upd = w > bw
                return (jnp.where(upd, cand, bc), jnp.maximum(bw, w))

            best_c, best_w = lax.fori_loop(1, C, tail_col, (best_c, best_w))
            row = CHUNKS_LAST * CHUNK_ROWS + g * LANES
            ow_buf[pl.ds(row, LANES)] = best_w
            oc_buf[pl.ds(row, LANES)] = best_c

    pltpu.sync_copy(ow_buf.at[pl.ds(0, ROWS_LAST)],
                    out_w_hbm.at[pl.ds(wbase, ROWS_LAST)])
    pltpu.sync_copy(oc_buf.at[pl.ds(0, ROWS_LAST)],
                    out_c_hbm.at[pl.ds(wbase, ROWS_LAST)])

    @pl.when(jnp.logical_not(last))
    def _():
        extra = ROWS_MAIN - ROWS_LAST
        pltpu.sync_copy(ow_buf.at[pl.ds(ROWS_LAST, extra)],
                        out_w_hbm.at[pl.ds(wbase + ROWS_LAST, extra)])
        pltpu.sync_copy(oc_buf.at[pl.ds(ROWS_LAST, extra)],
                        out_c_hbm.at[pl.ds(wbase + ROWS_LAST, extra)])


def kernel(weights, candidates):
    w = weights.reshape(N).astype(jnp.float32)
    cand = candidates.astype(jnp.int32)
    cand_t = cand.T
    tail = cand[N - TAIL_ROWS:, :].reshape(TAIL_ROWS * C)
    max_w, max_c = _sc_max_select(w, cand_t, tail)
    return (max_w.reshape(N, 1), max_c.astype(candidates.dtype))
